# baseline (speedup 1.0000x reference)
import jax
import jax.numpy as jnp
from jax import lax
from jax.experimental import pallas as pl
from jax.experimental.pallas import tpu as pltpu
from jax.experimental.pallas import tpu_sc as plsc

_NC = 2
_NS = 16
_NW = _NC * _NS
_C = 4096
_D = 12


def _sc_body(idx_hbm, tab_hbm, out_hbm, ids_v, tab_v, buf_v):
    per_w = idx_hbm.shape[0] // _NW
    wid = lax.axis_index("s") * _NC + lax.axis_index("c")
    base = wid * per_w
    lane = lax.broadcasted_iota(jnp.int32, (16,), 0)
    lane12 = lane * _D

    pltpu.sync_copy(tab_hbm, tab_v)

    @pl.loop(0, per_w // _C)
    def chunk_body(c):
        n0 = base + c * _C
        pltpu.sync_copy(idx_hbm.at[pl.ds(n0, _C)], ids_v)

        @pl.loop(0, _C // 16, unroll=8)
        def vec_body(t):
            iv = ids_v[pl.ds(t * 16, 16)]
            addr0 = iv * _D
            pos0 = lane12 + t * (16 * _D)
            for r in range(_D):
                val = plsc.load_gather(tab_v, [addr0 + r])
                p = pos0 + r
                plsc.store_scatter(buf_v, [p >> 7, p & 127], val)

        row0 = pl.multiple_of(n0 * _D // 128, 8)
        pltpu.sync_copy(buf_v, out_hbm.at[pl.ds(row0, _C * _D // 128)])


def _sc_lookup(idx_flat, tab_flat):
    n_total = idx_flat.shape[0]
    mesh = plsc.VectorSubcoreMesh(core_axis_name="c", subcore_axis_name="s")
    return pl.kernel(
        _sc_body,
        out_type=jax.ShapeDtypeStruct((n_total * _D // 128, 128), jnp.float32),
        mesh=mesh,
        compiler_params=pltpu.CompilerParams(needs_layout_passes=False),
        scratch_types=[
            pltpu.VMEM((_C,), jnp.int32),
            pltpu.VMEM((128,), jnp.float32),
            pltpu.VMEM((_C * _D // 128, 128), jnp.float32),
        ],
    )(idx_flat, tab_flat)


def kernel(inputs, table):
    n_rows, n_cols = inputs.shape
    idx_flat = inputs.reshape(-1)
    tab_flat = jnp.pad(table.reshape(-1), (0, 128 - table.size))
    out2 = _sc_lookup(idx_flat, tab_flat)
    return out2.reshape(n_rows, n_cols, table.shape[1])


# speedup vs baseline: 1.3665x; 1.3665x over previous
import functools

import jax
import jax.numpy as jnp
from jax import lax
from jax.experimental import pallas as pl
from jax.experimental.pallas import tpu as pltpu
from jax.experimental.pallas import tpu_sc as plsc

_NC = 2
_NS = 16
_NW = _NC * _NS
_C = 800


def _sc_body(idx_hbm, tab_hbm, out_hbm, idx_v, tab_v, buf_v):
    per_w = idx_hbm.shape[0] // _NW
    wid = lax.axis_index("s") * _NC + lax.axis_index("c")
    base = wid * per_w
    lane = lax.broadcasted_iota(jnp.int32, (16,), 0)

    pltpu.sync_copy(tab_hbm, tab_v)

    def chunk_body(c, carry):
        n0 = base + c * _C
        pltpu.sync_copy(idx_hbm.at[pl.ds(n0, _C)], idx_v)

        def vec_body(t, carry2):
            iv = idx_v[pl.ds(t * 16, 16)]
            addr0 = iv * 12
            row = t * 16 + lane
            for r in range(12):
                val = plsc.load_gather(tab_v, [addr0 + r])
                col = jnp.full((16,), r, jnp.int32)
                plsc.store_scatter(buf_v, [row, col], val)
            return carry2

        lax.fori_loop(0, _C // 16, vec_body, 0)
        pltpu.sync_copy(buf_v, out_hbm.at[pl.ds(n0, _C)])
        return carry

    lax.fori_loop(0, per_w // _C, chunk_body, 0)


def _sc_lookup(idx_flat, tab_flat):
    n_total = idx_flat.shape[0]
    mesh = plsc.VectorSubcoreMesh(core_axis_name="c", subcore_axis_name="s")
    return pl.kernel(
        _sc_body,
        out_type=jax.ShapeDtypeStruct((n_total, 12), jnp.float32),
        mesh=mesh,
        compiler_params=pltpu.CompilerParams(needs_layout_passes=False),
        scratch_types=[
            pltpu.VMEM((_C,), jnp.int32),
            pltpu.VMEM((128,), jnp.float32),
            pltpu.VMEM((_C, 12), jnp.float32),
        ],
    )(idx_flat, tab_flat)


def kernel(inputs, table):
    n_rows, n_cols = inputs.shape
    idx_flat = inputs.reshape(-1)
    tab_flat = jnp.pad(table.reshape(-1), (0, 128 - table.size))
    out2 = _sc_lookup(idx_flat, tab_flat)
    return out2.reshape(n_rows, n_cols, table.shape[1])
